# Initial kernel scaffold; baseline (speedup 1.0000x reference)
import jax
import jax.numpy as jnp
from jax.experimental import pallas as pl


def kernel(x, edge_index, y, train_mask, weights_layer, W1, b1, W2, b2, W3, b3, W4, b4):
    raise NotImplementedError("write your pallas kernel here")


# speedup vs baseline: 1.0000x; 1.0000x over previous
import jax
import jax.numpy as jnp
from jax.experimental import pallas as pl


def kernel(x, edge_index, y, train_mask, weights_layer, W1, b1, W2, b2, W3, b3, W4, b4):
    num_nodes = x.shape[0]
    loops = jnp.arange(num_nodes, dtype=edge_index.dtype)
    ei = jnp.concatenate([edge_index, jnp.stack([loops, loops], axis=0)], axis=1)
    row, col = ei[0], ei[1]
    deg = jnp.zeros((num_nodes,), dtype=jnp.float32).at[col].add(weights_layer)
    deg_inv = jnp.where(deg == 0, 0.0, 1.0 / deg)
    edge_weight = deg_inv[col] * weights_layer

    def gcn(h, W, b):
        h = h @ W + b
        msg = edge_weight[:, None] * h[row]
        return jnp.zeros((num_nodes, W.shape[1]), dtype=h.dtype).at[col].add(msg)

    h = jax.nn.relu(gcn(x, W1, b1))
    h = jax.nn.relu(gcn(h, W2, b2))
    h = jax.nn.relu(gcn(h, W3, b3))
    logits = gcn(h, W4, b4)
    lab = jax.nn.one_hot(y, 16, dtype=jnp.float32) * train_mask.astype(jnp.float32)[:, None]
    for _ in range(5):
        msg = edge_weight[:, None] * lab[row]
        lab = jnp.zeros((num_nodes, 16), dtype=jnp.float32).at[col].add(msg)
    return (jax.nn.log_softmax(logits, axis=1), jax.nn.log_softmax(lab, axis=1))


# speedup vs baseline: 9.8944x; 9.8941x over previous
import functools

import jax
import jax.numpy as jnp
from jax import lax
from jax.experimental import pallas as pl
from jax.experimental.pallas import tpu as pltpu
from jax.experimental.pallas import tpu_sc as plsc

N_NODES = 10000
NC, NS, LANES = 2, 16, 16
NW = NC * NS
BLK_E = 512
SUB = BLK_E // 128
CHUNK = 1000
NWRITE = N_NODES // CHUNK


def _splat(vec16, i):
    idx = jnp.full((LANES,), i, dtype=jnp.int32)
    return vec16.at[idx].get(mode="promise_in_bounds")


def _mesh():
    return plsc.VectorSubcoreMesh(
        core_axis_name="c", subcore_axis_name="s", num_cores=NC, num_subcores=NS
    )


_SC_PARAMS = pltpu.CompilerParams(
    use_tc_tiling_on_sc=False, needs_layout_passes=False
)


def _make_deg_kernel(ne_pad):
    per_w = ne_pad // NW
    blks = per_w // BLK_E

    @functools.partial(
        pl.kernel,
        out_type=jax.ShapeDtypeStruct((NC * N_NODES,), jnp.float32),
        mesh=_mesh(),
        compiler_params=_SC_PARAMS,
        scratch_types=[
            pltpu.VMEM((SUB, 128), jnp.int32),
            pltpu.VMEM((BLK_E,), jnp.float32),
            pltpu.VMEM_SHARED((N_NODES,), jnp.float32),
        ],
    )
    def deg_kernel(col2_hbm, w_hbm, z1_hbm, degp_hbm, idxc_v, w_v, dacc):
        cid = lax.axis_index("c")
        sid = lax.axis_index("s")
        wid = sid * NC + cid

        @pl.when(sid < NWRITE)
        def _zero():
            pltpu.sync_copy(z1_hbm, dacc.at[pl.ds(sid * CHUNK, CHUNK)])

        plsc.subcore_barrier()

        def blk(k, carry):
            base_r = pl.multiple_of(wid * (per_w // 128) + k * SUB, SUB)
            base_w = pl.multiple_of(wid * per_w + k * BLK_E, BLK_E)
            pltpu.sync_copy(col2_hbm.at[pl.ds(base_r, SUB)], idxc_v)
            pltpu.sync_copy(w_hbm.at[pl.ds(base_w, BLK_E)], w_v)
            for j in range(SUB):
                pltpu.sync_copy(
                    w_v.at[pl.ds(j * 128, 128)],
                    dacc.at[idxc_v.at[j]],
                    add=True,
                )
            return carry

        lax.fori_loop(0, blks, blk, 0)
        plsc.subcore_barrier()

        @pl.when(sid < NWRITE)
        def _write():
            r0 = sid * CHUNK
            o0 = pl.multiple_of(cid * N_NODES + r0, 8)
            pltpu.sync_copy(dacc.at[pl.ds(r0, CHUNK)], degp_hbm.at[pl.ds(o0, CHUNK)])

    return deg_kernel


def _make_prop_kernel(ne_pad, feat_w):
    per_w = ne_pad // NW
    blks = per_w // BLK_E
    nchunks = feat_w // LANES

    @functools.partial(
        pl.kernel,
        out_type=jax.ShapeDtypeStruct((NC, N_NODES, feat_w), jnp.float32),
        mesh=_mesh(),
        compiler_params=_SC_PARAMS,
        scratch_types=[
            pltpu.VMEM((SUB, 128), jnp.int32),
            pltpu.VMEM((SUB, 128), jnp.int32),
            pltpu.VMEM((BLK_E,), jnp.float32),
            pltpu.VMEM((BLK_E, feat_w), jnp.float32),
            pltpu.VMEM((CHUNK, feat_w), jnp.float32),
            pltpu.VMEM((CHUNK,), jnp.float32),
            pltpu.VMEM_SHARED((N_NODES, feat_w), jnp.float32),
            pltpu.SemaphoreType.DMA,
        ],
    )
    def prop_kernel(
        row2_hbm, col2_hbm, w_hbm, feat_hbm, dinv_hbm, zf_hbm, acc_hbm,
        idxr_v, idxc_v, w_v, rows_v, stage_v, dv_v, acc_s, sem,
    ):
        cid = lax.axis_index("c")
        sid = lax.axis_index("s")
        wid = sid * NC + cid

        @pl.when(sid < NWRITE)
        def _zero():
            pltpu.sync_copy(zf_hbm, acc_s.at[pl.ds(sid * CHUNK, CHUNK)])

        plsc.subcore_barrier()

        def blk(k, carry):
            base_r = pl.multiple_of(wid * (per_w // 128) + k * SUB, SUB)
            base_w = pl.multiple_of(wid * per_w + k * BLK_E, BLK_E)
            pltpu.sync_copy(row2_hbm.at[pl.ds(base_r, SUB)], idxr_v)
            pltpu.sync_copy(col2_hbm.at[pl.ds(base_r, SUB)], idxc_v)
            pltpu.sync_copy(w_hbm.at[pl.ds(base_w, BLK_E)], w_v)
            for j in range(SUB):
                pltpu.async_copy(
                    feat_hbm.at[idxr_v.at[j]],
                    rows_v.at[pl.ds(j * 128, 128)],
                    sem,
                ).wait()

            def grp(g, c2):
                w16 = w_v[pl.ds(g * LANES, LANES)]
                for t in range(LANES):
                    ws = _splat(w16, t)
                    e = g * LANES + t
                    for c in range(nchunks):
                        sl = pl.ds(c * LANES, LANES)
                        rows_v[e, sl] = rows_v[e, sl] * ws
                return c2

            lax.fori_loop(0, BLK_E // LANES, grp, 0)
            for j in range(SUB):
                pltpu.sync_copy(
                    rows_v.at[pl.ds(j * 128, 128)],
                    acc_s.at[idxc_v.at[j]],
                    add=True,
                )
            return carry

        lax.fori_loop(0, blks, blk, 0)
        plsc.subcore_barrier()

        @pl.when(sid < NWRITE)
        def _write():
            r0 = sid * CHUNK
            pltpu.sync_copy(acc_s.at[pl.ds(r0, CHUNK)], stage_v)
            pltpu.sync_copy(dinv_hbm.at[pl.ds(r0, CHUNK)], dv_v)

            def rowscale(r, c2):
                ws = plsc.load_gather(dv_v, [jnp.full((LANES,), r, jnp.int32)])
                for c in range(nchunks):
                    sl = pl.ds(c * LANES, LANES)
                    stage_v[r, sl] = stage_v[r, sl] * ws
                return c2

            lax.fori_loop(0, CHUNK, rowscale, 0)
            pltpu.sync_copy(stage_v, acc_hbm.at[cid, pl.ds(r0, CHUNK)])

    return prop_kernel


def _t0_body(x_ref, w1_ref, b1_ref, y_ref, m_ref, degp_ref, f0_ref, dinv_ref):
    h = jnp.dot(x_ref[...], w1_ref[...], preferred_element_type=jnp.float32)
    h = h + b1_ref[...]
    oh = (y_ref[...] == lax.broadcasted_iota(jnp.int32, (1, 16), 1)).astype(
        jnp.float32
    ) * m_ref[...]
    f0_ref[...] = jnp.concatenate([h, oh], axis=1)
    d = degp_ref[0] + degp_ref[1]
    dinv_ref[...] = jnp.where(d == 0.0, 0.0, 1.0 / d)


def _tmid_body(accp_ref, w_ref, b_ref, f_ref):
    g = accp_ref[0] + accp_ref[1]
    h = jnp.maximum(g[:, :32], 0.0)
    z = jnp.dot(h, w_ref[...], preferred_element_type=jnp.float32) + b_ref[...]
    f_ref[...] = jnp.concatenate([z, g[:, 32:]], axis=1)


def _log_softmax(x):
    m = jnp.max(x, axis=1, keepdims=True)
    s = x - m
    return s - jnp.log(jnp.sum(jnp.exp(s), axis=1, keepdims=True))


def _t4_body(accp_ref, out1_ref, f4_ref):
    g = accp_ref[0] + accp_ref[1]
    out1_ref[...] = _log_softmax(g[:, :16])
    f4_ref[...] = g[:, 16:]


def _t5_body(accp_ref, out2_ref):
    g = accp_ref[0] + accp_ref[1]
    out2_ref[...] = _log_softmax(g)


def kernel(x, edge_index, y, train_mask, weights_layer, W1, b1, W2, b2, W3, b3, W4, b4):
    n = x.shape[0]
    ne = edge_index.shape[1] + n
    ne_pad = ((ne + NW * BLK_E - 1) // (NW * BLK_E)) * (NW * BLK_E)

    loops = jnp.arange(n, dtype=jnp.int32)
    row = jnp.concatenate(
        [edge_index[0], loops, jnp.zeros((ne_pad - ne,), jnp.int32)]
    )
    col = jnp.concatenate(
        [edge_index[1], loops, jnp.zeros((ne_pad - ne,), jnp.int32)]
    )
    w = jnp.concatenate(
        [weights_layer, jnp.zeros((ne_pad - ne,), jnp.float32)]
    )
    row2 = row.reshape(-1, 128)
    col2 = col.reshape(-1, 128)
    z1 = jnp.zeros((CHUNK,), jnp.float32)
    z48 = jnp.zeros((CHUNK, 48), jnp.float32)
    z32 = jnp.zeros((CHUNK, 32), jnp.float32)
    z16 = jnp.zeros((CHUNK, 16), jnp.float32)
    y2 = y.reshape(n, 1).astype(jnp.int32)
    m2 = train_mask.reshape(n, 1).astype(jnp.float32)
    b1r = b1.reshape(1, -1)
    b2r = b2.reshape(1, -1)
    b3r = b3.reshape(1, -1)
    b4r = b4.reshape(1, -1)

    deg_k = _make_deg_kernel(ne_pad)
    prop48 = _make_prop_kernel(ne_pad, 48)
    prop32 = _make_prop_kernel(ne_pad, 32)
    prop16 = _make_prop_kernel(ne_pad, 16)

    degp = deg_k(col2, w, z1).reshape(NC, n)

    f32t = jnp.float32
    t0 = pl.pallas_call(
        _t0_body,
        out_shape=[
            jax.ShapeDtypeStruct((n, 48), f32t),
            jax.ShapeDtypeStruct((n,), f32t),
        ],
    )
    f0, dinv = t0(x, W1, b1r, y2, m2, degp)

    acc1 = prop48(row2, col2, w, f0, dinv, z48)
    t1 = pl.pallas_call(
        _tmid_body, out_shape=jax.ShapeDtypeStruct((n, 48), f32t)
    )
    f1 = t1(acc1, W2, b2r)

    acc2 = prop48(row2, col2, w, f1, dinv, z48)
    f2 = pl.pallas_call(
        _tmid_body, out_shape=jax.ShapeDtypeStruct((n, 48), f32t)
    )(acc2, W3, b3r)

    acc3 = prop48(row2, col2, w, f2, dinv, z48)
    f3 = pl.pallas_call(
        _tmid_body, out_shape=jax.ShapeDtypeStruct((n, 32), f32t)
    )(acc3, W4, b4r)

    acc4 = prop32(row2, col2, w, f3, dinv, z32)
    out1, f4 = pl.pallas_call(
        _t4_body,
        out_shape=[
            jax.ShapeDtypeStruct((n, 16), f32t),
            jax.ShapeDtypeStruct((n, 16), f32t),
        ],
    )(acc4)

    acc5 = prop16(row2, col2, w, f4, dinv, z16)
    out2 = pl.pallas_call(
        _t5_body, out_shape=jax.ShapeDtypeStruct((n, 16), f32t)
    )(acc5)

    return (out1, out2)
